# Initial kernel scaffold; baseline (speedup 1.0000x reference)
import functools

import jax
import jax.numpy as jnp
from jax import lax
from jax.experimental import pallas as pl
from jax.experimental.pallas import tpu as pltpu
from jax.experimental.pallas import tpu_sc as plsc

NUM_USERS = 10000
NUM_ITEMS = 40000
N = NUM_USERS + NUM_ITEMS
EMB = 64
E = 800000

NPAD = 50016
RPT = NPAD // 16
ZROWS = RPT // 3
EPAD = 819200
ER = EPAD // 128
ROWS_TILE = ER // 16
BLKS_TILE = ROWS_TILE // 8
DEG_ROWS_TILE = (ER // 2) // 16
DEG_BLKS_TILE = DEG_ROWS_TILE // 8


def _sc_scatter(x0, x1, x2, x3, srcr, dstr, zrows, ones,
                a0, a1, a2, a3, dga, dgb,
                src_v, dst_v, rows_v, ones_v, zbuf_v, gsem):
    c = lax.axis_index("c")
    s = lax.axis_index("s")

    pltpu.sync_copy(zrows, zbuf_v)
    pltpu.sync_copy(ones, ones_v)

    def zero_acc(acc):
        for k in range(3):
            pltpu.sync_copy(zbuf_v, acc.at[pl.ds(s * RPT + k * ZROWS, ZROWS)])

    def readback(acc, outref):
        pltpu.sync_copy(acc.at[pl.ds(s * RPT, RPT)],
                        outref.at[pl.ds(s * RPT, RPT)])

    def gather_block(xref, acc, r0):
        pltpu.sync_copy(srcr.at[pl.ds(r0, 8)], src_v)
        pltpu.sync_copy(dstr.at[pl.ds(r0, 8)], dst_v)
        cps = [pltpu.async_copy(xref.at[src_v.at[j]],
                                rows_v.at[pl.ds(j * 128, 128)], gsem)
               for j in range(8)]
        for cp in cps:
            cp.wait()
        for j in range(8):
            pltpu.sync_copy(rows_v.at[pl.ds(j * 128, 128)],
                            acc.at[dst_v.at[j]], add=True)

    def chunk_pass(xref, acc, outref):
        zero_acc(acc)
        plsc.subcore_barrier()
        base = s * ROWS_TILE

        def body(i, carry):
            gather_block(xref, acc, base + i * 8)
            return carry

        lax.fori_loop(0, BLKS_TILE, body, 0)
        plsc.subcore_barrier()
        readback(acc, outref)

    def deg_pass(acc, outref, lo):
        zero_acc(acc)
        plsc.subcore_barrier()
        base = lo + s * DEG_ROWS_TILE

        def body(i, carry):
            r0 = base + i * 8
            pltpu.sync_copy(dstr.at[pl.ds(r0, 8)], dst_v)
            for j in range(8):
                pltpu.sync_copy(ones_v, acc.at[dst_v.at[j]], add=True)
            return carry

        lax.fori_loop(0, DEG_BLKS_TILE, body, 0)
        plsc.subcore_barrier()
        readback(acc, outref)

    def run(acc):
        @pl.when(c == 0)
        def _():
            chunk_pass(x0, acc, a0)
            chunk_pass(x1, acc, a1)
            deg_pass(acc, dga, 0)

        @pl.when(c == 1)
        def _():
            chunk_pass(x2, acc, a2)
            chunk_pass(x3, acc, a3)
            deg_pass(acc, dgb, ER // 2)

    pl.run_scoped(run, pltpu.VMEM_SHARED((NPAD, 32), jnp.float32))


def _sc_call(x0, x1, x2, x3, srcr, dstr, zrows, ones):
    f32 = jnp.float32
    out_type = tuple(jax.ShapeDtypeStruct((NPAD, 32), f32) for _ in range(6))
    mesh = plsc.VectorSubcoreMesh(core_axis_name="c", subcore_axis_name="s")
    kern = pl.kernel(
        _sc_scatter, out_type=out_type, mesh=mesh,
        scratch_types=[
            pltpu.VMEM((8, 128), jnp.int32),
            pltpu.VMEM((8, 128), jnp.int32),
            pltpu.VMEM((1024, 32), f32),
            pltpu.VMEM((128, 32), f32),
            pltpu.VMEM((ZROWS, 32), f32),
            pltpu.SemaphoreType.DMA,
        ],
    )
    return kern(x0, x1, x2, x3, srcr, dstr, zrows, ones)


_BLK_I = 2000
_BLK_N = 2000


def _lrelu(x):
    return jnp.where(x >= 0, x, 0.01 * x)


def _proj_body(f0, f1, w0, b0, w1, b1, p0, p1):
    a0 = jnp.dot(f0[...], w0[...], preferred_element_type=jnp.float32) + b0[...]
    a1 = jnp.dot(f1[...], w1[...], preferred_element_type=jnp.float32) + b1[...]
    p0[...] = _lrelu(a0)
    p1[...] = _lrelu(a1)


def _project(feat0, feat1, W0, b0, W1, b1):
    D0 = feat0.shape[1]
    D1 = feat1.shape[1]
    grid = NUM_ITEMS // _BLK_I
    return pl.pallas_call(
        _proj_body,
        grid=(grid,),
        in_specs=[
            pl.BlockSpec((_BLK_I, D0), lambda i: (i, 0)),
            pl.BlockSpec((_BLK_I, D1), lambda i: (i, 0)),
            pl.BlockSpec((D0, EMB), lambda i: (0, 0)),
            pl.BlockSpec((1, EMB), lambda i: (0, 0)),
            pl.BlockSpec((D1, EMB), lambda i: (0, 0)),
            pl.BlockSpec((1, EMB), lambda i: (0, 0)),
        ],
        out_specs=[
            pl.BlockSpec((_BLK_I, EMB), lambda i: (i, 0)),
            pl.BlockSpec((_BLK_I, EMB), lambda i: (i, 0)),
        ],
        out_shape=[
            jax.ShapeDtypeStruct((NUM_ITEMS, EMB), jnp.float32),
            jax.ShapeDtypeStruct((NUM_ITEMS, EMB), jnp.float32),
        ],
    )(feat0, feat1, W0, b0.reshape(1, EMB), W1, b1.reshape(1, EMB))


def _comb_body(a0, a1, a2, a3, dga, dgb, ne, wg0, wg1, wid, out):
    deg = jnp.maximum(dga[:, 0:1] + dgb[:, 0:1], 1.0)
    agg0 = jnp.concatenate([a0[...], a1[...]], axis=1) / deg
    agg1 = jnp.concatenate([a2[...], a3[...]], axis=1) / deg
    nev = ne[...]
    idp = jnp.dot(nev, wid[...], preferred_element_type=jnp.float32)
    h0 = _lrelu(jnp.dot(agg0, wg0[...], preferred_element_type=jnp.float32) + idp)
    h1 = _lrelu(jnp.dot(agg1, wg1[...], preferred_element_type=jnp.float32) + idp)
    out[...] = h0 + h1 + nev


def _combine(a0, a1, a2, a3, dga, dgb, node_emb, Wg0, Wg1, Wid):
    grid = N // _BLK_N
    cspec = pl.BlockSpec((_BLK_N, 32), lambda i: (i, 0))
    wspec = pl.BlockSpec((EMB, EMB), lambda i: (0, 0))
    return pl.pallas_call(
        _comb_body,
        grid=(grid,),
        in_specs=[cspec, cspec, cspec, cspec, cspec, cspec,
                  pl.BlockSpec((_BLK_N, EMB), lambda i: (i, 0)),
                  wspec, wspec, wspec],
        out_specs=pl.BlockSpec((_BLK_N, EMB), lambda i: (i, 0)),
        out_shape=jax.ShapeDtypeStruct((N, EMB), jnp.float32),
    )(a0, a1, a2, a3, dga, dgb, node_emb, Wg0, Wg1, Wid)


def kernel(node_emb, feat0, feat1, user_pref0, user_pref1, edge_index,
           W_proj0, b_proj0, W_proj1, b_proj1, W_gcn0, W_gcn1, W_id):
    p0, p1 = _project(feat0, feat1, W_proj0, b_proj0, W_proj1, b_proj1)
    x0 = jnp.concatenate([user_pref0, p0], axis=0)
    x1 = jnp.concatenate([user_pref1, p1], axis=0)
    pres = jnp.stack([x0, x1])

    npad_e = EPAD - E
    src = jnp.concatenate([edge_index[0], jnp.zeros((npad_e,), jnp.int32)])
    dst = jnp.concatenate([edge_index[1],
                           jnp.full((npad_e,), N, jnp.int32)])
    srcr = src.reshape(ER, 128)
    dstr = dst.reshape(ER, 128)

    zrows = jnp.zeros((ZROWS, 32), jnp.float32)
    ones = jnp.ones((128, 32), jnp.float32)

    a0, a1, a2, a3, dga, dgb = _sc_call(
        x0[:, :32], x0[:, 32:], x1[:, :32], x1[:, 32:],
        srcr, dstr, zrows, ones)

    emb = _combine(a0, a1, a2, a3, dga, dgb, node_emb, W_gcn0, W_gcn1, W_id)
    return emb[:NUM_USERS], emb[NUM_USERS:], node_emb, pres


# speedup vs baseline: 3.7564x; 3.7564x over previous
import functools

import jax
import jax.numpy as jnp
from jax import lax
from jax.experimental import pallas as pl
from jax.experimental.pallas import tpu as pltpu
from jax.experimental.pallas import tpu_sc as plsc

NUM_USERS = 10000
NUM_ITEMS = 40000
N = NUM_USERS + NUM_ITEMS
EMB = 64
E = 800000

NPAD = 50176
RPT = NPAD // 16
ZROWS = RPT // 4
EPAD = 819200
ER = EPAD // 128
ROWS_TILE = ER // 16
BR = 4
BLKS_TILE = ROWS_TILE // BR
DEG_ROWS_TILE = (ER // 2) // 16
DEG_BLKS_TILE = DEG_ROWS_TILE // BR


def _sc_scatter(x0, x1, x2, x3, srcr, dstr, zrows, ones,
                a0, a1, a2, a3, dga, dgb,
                src_v, dst_v, rows_v, ones_v, acc, gsem):
    c = lax.axis_index("c")
    s = lax.axis_index("s")

    pltpu.sync_copy(ones, ones_v)

    def zero_acc(acc):
        pltpu.sync_copy(zrows.at[pl.ds(s * RPT, RPT)],
                        acc.at[pl.ds(s * RPT, RPT)])

    def readback(acc, outref):
        pltpu.sync_copy(acc.at[pl.ds(s * RPT, RPT)],
                        outref.at[pl.ds(s * RPT, RPT)])

    def gather_block(xref, acc, r0):
        pltpu.sync_copy(srcr.at[pl.ds(r0, BR)], src_v)
        pltpu.sync_copy(dstr.at[pl.ds(r0, BR)], dst_v)
        cps = [pltpu.async_copy(xref.at[src_v.at[j]],
                                rows_v.at[pl.ds(j * 128, 128)], gsem)
               for j in range(BR)]
        for cp in cps:
            cp.wait()
        for j in range(BR):
            pltpu.sync_copy(rows_v.at[pl.ds(j * 128, 128)],
                            acc.at[dst_v.at[j]], add=True)

    def chunk_pass(xref, acc, outref):
        zero_acc(acc)
        plsc.subcore_barrier()
        base = s * ROWS_TILE

        def body(i, carry):
            gather_block(xref, acc, base + i * BR)
            return carry

        lax.fori_loop(0, BLKS_TILE, body, 0)
        plsc.subcore_barrier()
        readback(acc, outref)

    def deg_pass(acc, outref, lo):
        zero_acc(acc)
        plsc.subcore_barrier()
        base = lo + s * DEG_ROWS_TILE

        def body(i, carry):
            r0 = base + i * BR
            pltpu.sync_copy(dstr.at[pl.ds(r0, BR)], dst_v)
            for j in range(BR):
                pltpu.sync_copy(ones_v, acc.at[dst_v.at[j]], add=True)
            return carry

        lax.fori_loop(0, DEG_BLKS_TILE, body, 0)
        plsc.subcore_barrier()
        readback(acc, outref)

    @pl.when(c == 0)
    def _():
        chunk_pass(x0, acc, a0)
        chunk_pass(x1, acc, a1)
        deg_pass(acc, dga, 0)

    @pl.when(c == 1)
    def _():
        chunk_pass(x2, acc, a2)
        chunk_pass(x3, acc, a3)
        deg_pass(acc, dgb, ER // 2)


def _sc_call(x0, x1, x2, x3, srcr, dstr, zrows, ones):
    f32 = jnp.float32
    out_type = tuple(jax.ShapeDtypeStruct((NPAD, 32), f32) for _ in range(6))
    mesh = plsc.VectorSubcoreMesh(core_axis_name="c", subcore_axis_name="s")
    kern = pl.kernel(
        _sc_scatter, out_type=out_type, mesh=mesh,
        compiler_params=pltpu.CompilerParams(use_tc_tiling_on_sc=False),
        scratch_types=[
            pltpu.VMEM((BR, 128), jnp.int32),
            pltpu.VMEM((BR, 128), jnp.int32),
            pltpu.VMEM((BR * 128, 32), f32),
            pltpu.VMEM((128, 32), f32),
            pltpu.VMEM_SHARED((NPAD, 32), f32),
            pltpu.SemaphoreType.DMA,
        ],
    )
    return kern(x0, x1, x2, x3, srcr, dstr, zrows, ones)


_BLK_I = 2000
_BLK_N = 2000


def _lrelu(x):
    return jnp.where(x >= 0, x, 0.01 * x)


def _proj_body(f0, f1, w0, b0, w1, b1, p0, p1):
    a0 = jnp.dot(f0[...], w0[...], preferred_element_type=jnp.float32) + b0[...]
    a1 = jnp.dot(f1[...], w1[...], preferred_element_type=jnp.float32) + b1[...]
    p0[...] = _lrelu(a0)
    p1[...] = _lrelu(a1)


def _project(feat0, feat1, W0, b0, W1, b1):
    D0 = feat0.shape[1]
    D1 = feat1.shape[1]
    grid = NUM_ITEMS // _BLK_I
    return pl.pallas_call(
        _proj_body,
        grid=(grid,),
        in_specs=[
            pl.BlockSpec((_BLK_I, D0), lambda i: (i, 0)),
            pl.BlockSpec((_BLK_I, D1), lambda i: (i, 0)),
            pl.BlockSpec((D0, EMB), lambda i: (0, 0)),
            pl.BlockSpec((1, EMB), lambda i: (0, 0)),
            pl.BlockSpec((D1, EMB), lambda i: (0, 0)),
            pl.BlockSpec((1, EMB), lambda i: (0, 0)),
        ],
        out_specs=[
            pl.BlockSpec((_BLK_I, EMB), lambda i: (i, 0)),
            pl.BlockSpec((_BLK_I, EMB), lambda i: (i, 0)),
        ],
        out_shape=[
            jax.ShapeDtypeStruct((NUM_ITEMS, EMB), jnp.float32),
            jax.ShapeDtypeStruct((NUM_ITEMS, EMB), jnp.float32),
        ],
    )(feat0, feat1, W0, b0.reshape(1, EMB), W1, b1.reshape(1, EMB))


def _comb_body(a0, a1, a2, a3, dga, dgb, ne, wg0, wg1, wid, out):
    deg = jnp.maximum(dga[:, 0:1] + dgb[:, 0:1], 1.0)
    agg0 = jnp.concatenate([a0[...], a1[...]], axis=1) / deg
    agg1 = jnp.concatenate([a2[...], a3[...]], axis=1) / deg
    nev = ne[...]
    idp = jnp.dot(nev, wid[...], preferred_element_type=jnp.float32)
    h0 = _lrelu(jnp.dot(agg0, wg0[...], preferred_element_type=jnp.float32) + idp)
    h1 = _lrelu(jnp.dot(agg1, wg1[...], preferred_element_type=jnp.float32) + idp)
    out[...] = h0 + h1 + nev


def _combine(a0, a1, a2, a3, dga, dgb, node_emb, Wg0, Wg1, Wid):
    grid = N // _BLK_N
    cspec = pl.BlockSpec((_BLK_N, 32), lambda i: (i, 0))
    wspec = pl.BlockSpec((EMB, EMB), lambda i: (0, 0))
    return pl.pallas_call(
        _comb_body,
        grid=(grid,),
        in_specs=[cspec, cspec, cspec, cspec, cspec, cspec,
                  pl.BlockSpec((_BLK_N, EMB), lambda i: (i, 0)),
                  wspec, wspec, wspec],
        out_specs=pl.BlockSpec((_BLK_N, EMB), lambda i: (i, 0)),
        out_shape=jax.ShapeDtypeStruct((N, EMB), jnp.float32),
    )(a0, a1, a2, a3, dga, dgb, node_emb, Wg0, Wg1, Wid)


def kernel(node_emb, feat0, feat1, user_pref0, user_pref1, edge_index,
           W_proj0, b_proj0, W_proj1, b_proj1, W_gcn0, W_gcn1, W_id):
    p0, p1 = _project(feat0, feat1, W_proj0, b_proj0, W_proj1, b_proj1)
    x0 = jnp.concatenate([user_pref0, p0], axis=0)
    x1 = jnp.concatenate([user_pref1, p1], axis=0)
    pres = jnp.stack([x0, x1])

    npad_e = EPAD - E
    src = jnp.concatenate([edge_index[0], jnp.zeros((npad_e,), jnp.int32)])
    dst = jnp.concatenate([edge_index[1],
                           jnp.full((npad_e,), N, jnp.int32)])
    srcr = src.reshape(ER, 128)
    dstr = dst.reshape(ER, 128)

    zrows = jnp.zeros((NPAD, 32), jnp.float32)
    ones = jnp.ones((128, 32), jnp.float32)

    a0, a1, a2, a3, dga, dgb = _sc_call(
        x0[:, :32], x0[:, 32:], x1[:, :32], x1[:, 32:],
        srcr, dstr, zrows, ones)

    emb = _combine(a0, a1, a2, a3, dga, dgb, node_emb, W_gcn0, W_gcn1, W_id)
    return emb[:NUM_USERS], emb[NUM_USERS:], node_emb, pres


# speedup vs baseline: 4.5783x; 1.2188x over previous
import functools

import jax
import jax.numpy as jnp
from jax import lax
from jax.experimental import pallas as pl
from jax.experimental.pallas import tpu as pltpu
from jax.experimental.pallas import tpu_sc as plsc

NUM_USERS = 10000
NUM_ITEMS = 40000
N = NUM_USERS + NUM_ITEMS
EMB = 64
E = 800000

NPAD = 50176
RPT = NPAD // 16
ZROWS = RPT // 4
EPAD = 819200
ER = EPAD // 128
ROWS_TILE = ER // 16
CH = 16
NCH = ROWS_TILE // CH
DEG_CH = 8
DEG_ROWS_TILE = (ER // 2) // 16
DEG_NCH = DEG_ROWS_TILE // DEG_CH
NBUF = 4


def _sc_scatter(x0, x1, x2, x3, srcr, dstr, zrows, ones,
                a0, a1, a2, a3, dga, dgb,
                src_v, dst_v, r0_v, r1_v, r2_v, r3_v, ones_v, acc,
                gsem, ssem):
    c = lax.axis_index("c")
    s = lax.axis_index("s")
    rows = [r0_v, r1_v, r2_v, r3_v]

    pltpu.sync_copy(ones, ones_v)

    def zero_acc(acc):
        pltpu.sync_copy(zrows.at[pl.ds(s * RPT, RPT)],
                        acc.at[pl.ds(s * RPT, RPT)])

    def readback(acc, outref):
        pltpu.sync_copy(acc.at[pl.ds(s * RPT, RPT)],
                        outref.at[pl.ds(s * RPT, RPT)])

    def chunk_pass(xref, acc, outref):
        zero_acc(acc)
        plsc.subcore_barrier()
        base = s * ROWS_TILE

        def body(i, carry):
            r0 = base + i * CH
            pltpu.sync_copy(srcr.at[pl.ds(r0, CH)], src_v)
            pltpu.sync_copy(dstr.at[pl.ds(r0, CH)], dst_v)
            g = {}
            sc = {}

            def fire_scatter(b):
                g[b].wait()
                sc[b] = pltpu.async_copy(rows[b % NBUF],
                                         acc.at[dst_v.at[b]], ssem, add=True)

            for b in range(CH):
                if b >= NBUF:
                    sc[b - NBUF].wait()
                g[b] = pltpu.async_copy(xref.at[src_v.at[b]],
                                        rows[b % NBUF], gsem)
                if b >= 2:
                    fire_scatter(b - 2)
            for b in (CH - 2, CH - 1):
                fire_scatter(b)
            for b in range(CH - NBUF, CH):
                sc[b].wait()
            return carry

        lax.fori_loop(0, NCH, body, 0)
        plsc.subcore_barrier()
        readback(acc, outref)

    def deg_pass(acc, outref, lo):
        zero_acc(acc)
        plsc.subcore_barrier()
        base = lo + s * DEG_ROWS_TILE

        def body(i, carry):
            r0 = base + i * DEG_CH
            pltpu.sync_copy(dstr.at[pl.ds(r0, DEG_CH)], dst_v.at[pl.ds(0, DEG_CH)])
            sc = [pltpu.async_copy(ones_v, acc.at[dst_v.at[b]], ssem,
                                   add=True)
                  for b in range(DEG_CH)]
            for cp in sc:
                cp.wait()
            return carry

        lax.fori_loop(0, DEG_NCH, body, 0)
        plsc.subcore_barrier()
        readback(acc, outref)

    @pl.when(c == 0)
    def _():
        chunk_pass(x0, acc, a0)
        chunk_pass(x1, acc, a1)
        deg_pass(acc, dga, 0)

    @pl.when(c == 1)
    def _():
        chunk_pass(x2, acc, a2)
        chunk_pass(x3, acc, a3)
        deg_pass(acc, dgb, ER // 2)


def _sc_call(x0, x1, x2, x3, srcr, dstr, zrows, ones):
    f32 = jnp.float32
    out_type = tuple(jax.ShapeDtypeStruct((NPAD, 32), f32) for _ in range(6))
    mesh = plsc.VectorSubcoreMesh(core_axis_name="c", subcore_axis_name="s")
    kern = pl.kernel(
        _sc_scatter, out_type=out_type, mesh=mesh,
        compiler_params=pltpu.CompilerParams(use_tc_tiling_on_sc=False),
        scratch_types=[
            pltpu.VMEM((CH, 128), jnp.int32),
            pltpu.VMEM((CH, 128), jnp.int32),
            pltpu.VMEM((128, 32), f32),
            pltpu.VMEM((128, 32), f32),
            pltpu.VMEM((128, 32), f32),
            pltpu.VMEM((128, 32), f32),
            pltpu.VMEM((128, 32), f32),
            pltpu.VMEM_SHARED((NPAD, 32), f32),
            pltpu.SemaphoreType.DMA,
            pltpu.SemaphoreType.DMA,
        ],
    )
    return kern(x0, x1, x2, x3, srcr, dstr, zrows, ones)


_BLK_I = 2000
_BLK_N = 2000


def _lrelu(x):
    return jnp.where(x >= 0, x, 0.01 * x)


def _proj_body(f0, f1, w0, b0, w1, b1, p0, p1):
    a0 = jnp.dot(f0[...], w0[...], preferred_element_type=jnp.float32) + b0[...]
    a1 = jnp.dot(f1[...], w1[...], preferred_element_type=jnp.float32) + b1[...]
    p0[...] = _lrelu(a0)
    p1[...] = _lrelu(a1)


def _project(feat0, feat1, W0, b0, W1, b1):
    D0 = feat0.shape[1]
    D1 = feat1.shape[1]
    grid = NUM_ITEMS // _BLK_I
    return pl.pallas_call(
        _proj_body,
        grid=(grid,),
        in_specs=[
            pl.BlockSpec((_BLK_I, D0), lambda i: (i, 0)),
            pl.BlockSpec((_BLK_I, D1), lambda i: (i, 0)),
            pl.BlockSpec((D0, EMB), lambda i: (0, 0)),
            pl.BlockSpec((1, EMB), lambda i: (0, 0)),
            pl.BlockSpec((D1, EMB), lambda i: (0, 0)),
            pl.BlockSpec((1, EMB), lambda i: (0, 0)),
        ],
        out_specs=[
            pl.BlockSpec((_BLK_I, EMB), lambda i: (i, 0)),
            pl.BlockSpec((_BLK_I, EMB), lambda i: (i, 0)),
        ],
        out_shape=[
            jax.ShapeDtypeStruct((NUM_ITEMS, EMB), jnp.float32),
            jax.ShapeDtypeStruct((NUM_ITEMS, EMB), jnp.float32),
        ],
    )(feat0, feat1, W0, b0.reshape(1, EMB), W1, b1.reshape(1, EMB))


def _comb_body(a0, a1, a2, a3, dga, dgb, ne, wg0, wg1, wid, out):
    deg = jnp.maximum(dga[:, 0:1] + dgb[:, 0:1], 1.0)
    agg0 = jnp.concatenate([a0[...], a1[...]], axis=1) / deg
    agg1 = jnp.concatenate([a2[...], a3[...]], axis=1) / deg
    nev = ne[...]
    idp = jnp.dot(nev, wid[...], preferred_element_type=jnp.float32)
    h0 = _lrelu(jnp.dot(agg0, wg0[...], preferred_element_type=jnp.float32) + idp)
    h1 = _lrelu(jnp.dot(agg1, wg1[...], preferred_element_type=jnp.float32) + idp)
    out[...] = h0 + h1 + nev


def _combine(a0, a1, a2, a3, dga, dgb, node_emb, Wg0, Wg1, Wid):
    grid = N // _BLK_N
    cspec = pl.BlockSpec((_BLK_N, 32), lambda i: (i, 0))
    wspec = pl.BlockSpec((EMB, EMB), lambda i: (0, 0))
    return pl.pallas_call(
        _comb_body,
        grid=(grid,),
        in_specs=[cspec, cspec, cspec, cspec, cspec, cspec,
                  pl.BlockSpec((_BLK_N, EMB), lambda i: (i, 0)),
                  wspec, wspec, wspec],
        out_specs=pl.BlockSpec((_BLK_N, EMB), lambda i: (i, 0)),
        out_shape=jax.ShapeDtypeStruct((N, EMB), jnp.float32),
    )(a0, a1, a2, a3, dga, dgb, node_emb, Wg0, Wg1, Wid)


def kernel(node_emb, feat0, feat1, user_pref0, user_pref1, edge_index,
           W_proj0, b_proj0, W_proj1, b_proj1, W_gcn0, W_gcn1, W_id):
    p0, p1 = _project(feat0, feat1, W_proj0, b_proj0, W_proj1, b_proj1)
    x0 = jnp.concatenate([user_pref0, p0], axis=0)
    x1 = jnp.concatenate([user_pref1, p1], axis=0)
    pres = jnp.stack([x0, x1])

    npad_e = EPAD - E
    src = jnp.concatenate([edge_index[0], jnp.zeros((npad_e,), jnp.int32)])
    dst = jnp.concatenate([edge_index[1],
                           jnp.full((npad_e,), N, jnp.int32)])
    srcr = src.reshape(ER, 128)
    dstr = dst.reshape(ER, 128)

    zrows = jnp.zeros((NPAD, 32), jnp.float32)
    ones = jnp.ones((128, 32), jnp.float32)

    a0, a1, a2, a3, dga, dgb = _sc_call(
        x0[:, :32], x0[:, 32:], x1[:, :32], x1[:, 32:],
        srcr, dstr, zrows, ones)

    emb = _combine(a0, a1, a2, a3, dga, dgb, node_emb, W_gcn0, W_gcn1, W_id)
    return emb[:NUM_USERS], emb[NUM_USERS:], node_emb, pres


# speedup vs baseline: 4.5946x; 1.0036x over previous
import functools

import jax
import jax.numpy as jnp
from jax import lax
from jax.experimental import pallas as pl
from jax.experimental.pallas import tpu as pltpu
from jax.experimental.pallas import tpu_sc as plsc

NUM_USERS = 10000
NUM_ITEMS = 40000
N = NUM_USERS + NUM_ITEMS
EMB = 64
E = 800000

NPAD = 50176
RPT = NPAD // 16
ZROWS = RPT // 4
EPAD = 819200
ER = EPAD // 128
ROWS_TILE = ER // 16
CH = 16
NCH = ROWS_TILE // CH
GB = CH // 2
DEG_CH = 8
DEG_ROWS_TILE = (ER // 2) // 16
DEG_NCH = DEG_ROWS_TILE // DEG_CH
NBUF = 3


def _sc_scatter(x0, x1, x2, x3, srcf, dstr, zrows, ones,
                a0, a1, a2, a3, dga, dgb,
                src_v, dst_v, r0_v, r1_v, r2_v, acc,
                gsem, ssem):
    c = lax.axis_index("c")
    s = lax.axis_index("s")
    rows = [r0_v, r1_v, r2_v]

    def zero_acc(acc):
        pltpu.sync_copy(zrows.at[pl.ds(s * RPT, RPT)],
                        acc.at[pl.ds(s * RPT, RPT)])

    def readback(acc, outref):
        pltpu.sync_copy(acc.at[pl.ds(s * RPT, RPT)],
                        outref.at[pl.ds(s * RPT, RPT)])

    def chunk_pass(xref, acc, outref):
        zero_acc(acc)
        plsc.subcore_barrier()
        base = s * ROWS_TILE

        def body(i, carry):
            r0 = base + i * CH
            pltpu.sync_copy(srcf.at[pl.ds(r0 * 128, CH * 128)], src_v)
            pltpu.sync_copy(dstr.at[pl.ds(r0, CH)], dst_v)
            g = {}
            sc = {}

            def fire_scatters(b):
                g[b].wait()
                buf = rows[b % NBUF]
                for h in range(2):
                    sc[(b, h)] = pltpu.async_copy(
                        buf.at[pl.ds(h * 128, 128)],
                        acc.at[dst_v.at[2 * b + h]], ssem, add=True)

            for b in range(GB):
                if b >= NBUF:
                    sc[(b - NBUF, 0)].wait()
                    sc[(b - NBUF, 1)].wait()
                g[b] = pltpu.async_copy(
                    xref.at[src_v.at[pl.ds(b * 256, 256)]],
                    rows[b % NBUF], gsem)
                if b >= 1:
                    fire_scatters(b - 1)
            fire_scatters(GB - 1)
            for b in range(GB - NBUF, GB):
                sc[(b, 0)].wait()
                sc[(b, 1)].wait()
            return carry

        lax.fori_loop(0, NCH, body, 0)
        plsc.subcore_barrier()
        readback(acc, outref)

    def deg_pass(acc, outref, lo):
        zero_acc(acc)
        pltpu.sync_copy(ones, r0_v.at[pl.ds(0, 128)])
        plsc.subcore_barrier()
        base = lo + s * DEG_ROWS_TILE

        def body(i, carry):
            r0 = base + i * DEG_CH
            pltpu.sync_copy(dstr.at[pl.ds(r0, DEG_CH)], dst_v.at[pl.ds(0, DEG_CH)])
            sc = [pltpu.async_copy(r0_v.at[pl.ds(0, 128)],
                                   acc.at[dst_v.at[b]], ssem, add=True)
                  for b in range(DEG_CH)]
            for cp in sc:
                cp.wait()
            return carry

        lax.fori_loop(0, DEG_NCH, body, 0)
        plsc.subcore_barrier()
        readback(acc, outref)

    @pl.when(c == 0)
    def _():
        chunk_pass(x0, acc, a0)
        chunk_pass(x1, acc, a1)
        deg_pass(acc, dga, 0)

    @pl.when(c == 1)
    def _():
        chunk_pass(x2, acc, a2)
        chunk_pass(x3, acc, a3)
        deg_pass(acc, dgb, ER // 2)


def _sc_call(x0, x1, x2, x3, srcr, dstr, zrows, ones):
    f32 = jnp.float32
    out_type = tuple(jax.ShapeDtypeStruct((NPAD, 32), f32) for _ in range(6))
    mesh = plsc.VectorSubcoreMesh(core_axis_name="c", subcore_axis_name="s")
    kern = pl.kernel(
        _sc_scatter, out_type=out_type, mesh=mesh,
        compiler_params=pltpu.CompilerParams(use_tc_tiling_on_sc=False),
        scratch_types=[
            pltpu.VMEM((CH * 128,), jnp.int32),
            pltpu.VMEM((CH, 128), jnp.int32),
            pltpu.VMEM((256, 32), f32),
            pltpu.VMEM((256, 32), f32),
            pltpu.VMEM((256, 32), f32),
            pltpu.VMEM_SHARED((NPAD, 32), f32),
            pltpu.SemaphoreType.DMA,
            pltpu.SemaphoreType.DMA,
        ],
    )
    return kern(x0, x1, x2, x3, srcr, dstr, zrows, ones)


_BLK_I = 2000
_BLK_N = 2000


def _lrelu(x):
    return jnp.where(x >= 0, x, 0.01 * x)


def _proj_body(f0, f1, w0, b0, w1, b1, p0, p1):
    a0 = jnp.dot(f0[...], w0[...], preferred_element_type=jnp.float32) + b0[...]
    a1 = jnp.dot(f1[...], w1[...], preferred_element_type=jnp.float32) + b1[...]
    p0[...] = _lrelu(a0)
    p1[...] = _lrelu(a1)


def _project(feat0, feat1, W0, b0, W1, b1):
    D0 = feat0.shape[1]
    D1 = feat1.shape[1]
    grid = NUM_ITEMS // _BLK_I
    return pl.pallas_call(
        _proj_body,
        grid=(grid,),
        in_specs=[
            pl.BlockSpec((_BLK_I, D0), lambda i: (i, 0)),
            pl.BlockSpec((_BLK_I, D1), lambda i: (i, 0)),
            pl.BlockSpec((D0, EMB), lambda i: (0, 0)),
            pl.BlockSpec((1, EMB), lambda i: (0, 0)),
            pl.BlockSpec((D1, EMB), lambda i: (0, 0)),
            pl.BlockSpec((1, EMB), lambda i: (0, 0)),
        ],
        out_specs=[
            pl.BlockSpec((_BLK_I, EMB), lambda i: (i, 0)),
            pl.BlockSpec((_BLK_I, EMB), lambda i: (i, 0)),
        ],
        out_shape=[
            jax.ShapeDtypeStruct((NUM_ITEMS, EMB), jnp.float32),
            jax.ShapeDtypeStruct((NUM_ITEMS, EMB), jnp.float32),
        ],
    )(feat0, feat1, W0, b0.reshape(1, EMB), W1, b1.reshape(1, EMB))


def _comb_body(a0, a1, a2, a3, dga, dgb, ne, wg0, wg1, wid, out):
    deg = jnp.maximum(dga[:, 0:1] + dgb[:, 0:1], 1.0)
    agg0 = jnp.concatenate([a0[...], a1[...]], axis=1) / deg
    agg1 = jnp.concatenate([a2[...], a3[...]], axis=1) / deg
    nev = ne[...]
    idp = jnp.dot(nev, wid[...], preferred_element_type=jnp.float32)
    h0 = _lrelu(jnp.dot(agg0, wg0[...], preferred_element_type=jnp.float32) + idp)
    h1 = _lrelu(jnp.dot(agg1, wg1[...], preferred_element_type=jnp.float32) + idp)
    out[...] = h0 + h1 + nev


def _combine(a0, a1, a2, a3, dga, dgb, node_emb, Wg0, Wg1, Wid):
    grid = N // _BLK_N
    cspec = pl.BlockSpec((_BLK_N, 32), lambda i: (i, 0))
    wspec = pl.BlockSpec((EMB, EMB), lambda i: (0, 0))
    return pl.pallas_call(
        _comb_body,
        grid=(grid,),
        in_specs=[cspec, cspec, cspec, cspec, cspec, cspec,
                  pl.BlockSpec((_BLK_N, EMB), lambda i: (i, 0)),
                  wspec, wspec, wspec],
        out_specs=pl.BlockSpec((_BLK_N, EMB), lambda i: (i, 0)),
        out_shape=jax.ShapeDtypeStruct((N, EMB), jnp.float32),
    )(a0, a1, a2, a3, dga, dgb, node_emb, Wg0, Wg1, Wid)


def kernel(node_emb, feat0, feat1, user_pref0, user_pref1, edge_index,
           W_proj0, b_proj0, W_proj1, b_proj1, W_gcn0, W_gcn1, W_id):
    p0, p1 = _project(feat0, feat1, W_proj0, b_proj0, W_proj1, b_proj1)
    x0 = jnp.concatenate([user_pref0, p0], axis=0)
    x1 = jnp.concatenate([user_pref1, p1], axis=0)
    pres = jnp.stack([x0, x1])

    npad_e = EPAD - E
    src = jnp.concatenate([edge_index[0], jnp.zeros((npad_e,), jnp.int32)])
    dst = jnp.concatenate([edge_index[1],
                           jnp.full((npad_e,), N, jnp.int32)])
    dstr = dst.reshape(ER, 128)

    zrows = jnp.zeros((NPAD, 32), jnp.float32)
    ones = jnp.ones((128, 32), jnp.float32)

    a0, a1, a2, a3, dga, dgb = _sc_call(
        x0[:, :32], x0[:, 32:], x1[:, :32], x1[:, 32:],
        src, dstr, zrows, ones)

    emb = _combine(a0, a1, a2, a3, dga, dgb, node_emb, W_gcn0, W_gcn1, W_id)
    return emb[:NUM_USERS], emb[NUM_USERS:], node_emb, pres


# speedup vs baseline: 4.6446x; 1.0109x over previous
import functools

import jax
import jax.numpy as jnp
from jax import lax
from jax.experimental import pallas as pl
from jax.experimental.pallas import tpu as pltpu
from jax.experimental.pallas import tpu_sc as plsc

NUM_USERS = 10000
NUM_ITEMS = 40000
N = NUM_USERS + NUM_ITEMS
EMB = 64
E = 800000

NPAD = 50176
RPT = NPAD // 16
ZROWS = RPT // 4
EPAD = 819200
ER = EPAD // 128
ROWS_TILE = ER // 16
CH = 16
NCH = ROWS_TILE // CH
DEG_CH = 8
DEG_ROWS_TILE = (ER // 2) // 16
DEG_NCH = DEG_ROWS_TILE // DEG_CH
NBUF = 6


def _sc_scatter(x0, x1, x2, x3, srcf, dstr, zrows, ones,
                a0, a1, a2, a3, dga, dgb,
                src_v, dst_v, r0_v, r1_v, r2_v, r3_v, r4_v, r5_v, acc,
                gsem, ssem):
    c = lax.axis_index("c")
    s = lax.axis_index("s")
    rows = [r0_v, r1_v, r2_v, r3_v, r4_v, r5_v]

    def zero_acc(acc):
        pltpu.sync_copy(zrows.at[pl.ds(s * RPT, RPT)],
                        acc.at[pl.ds(s * RPT, RPT)])

    def readback(acc, outref):
        pltpu.sync_copy(acc.at[pl.ds(s * RPT, RPT)],
                        outref.at[pl.ds(s * RPT, RPT)])

    def chunk_pass(xref, acc, outref):
        zero_acc(acc)
        plsc.subcore_barrier()
        base = s * ROWS_TILE

        def body(i, carry):
            r0 = base + i * CH
            pltpu.sync_copy(srcf.at[pl.ds(r0 * 128, CH * 128)], src_v)
            pltpu.sync_copy(dstr.at[pl.ds(r0, CH)], dst_v)
            g = {}
            sc = {}

            def fire_scatter(b):
                g[b].wait()
                sc[b] = pltpu.async_copy(rows[b % NBUF],
                                         acc.at[dst_v.at[b]], ssem, add=True)

            for b in range(CH):
                if b >= NBUF:
                    sc[b - NBUF].wait()
                g[b] = pltpu.async_copy(
                    xref.at[src_v.at[pl.ds(b * 128, 128)]],
                    rows[b % NBUF], gsem)
                if b >= NBUF - 1:
                    fire_scatter(b - (NBUF - 1))
            for b in range(CH - (NBUF - 1), CH):
                fire_scatter(b)
            for b in range(CH - NBUF, CH):
                sc[b].wait()
            return carry

        lax.fori_loop(0, NCH, body, 0)
        plsc.subcore_barrier()
        readback(acc, outref)

    def deg_pass(acc, outref, lo):
        zero_acc(acc)
        pltpu.sync_copy(ones, r0_v.at[pl.ds(0, 128)])
        plsc.subcore_barrier()
        base = lo + s * DEG_ROWS_TILE

        def body(i, carry):
            r0 = base + i * DEG_CH
            pltpu.sync_copy(dstr.at[pl.ds(r0, DEG_CH)], dst_v.at[pl.ds(0, DEG_CH)])
            sc = [pltpu.async_copy(r0_v.at[pl.ds(0, 128)],
                                   acc.at[dst_v.at[b]], ssem, add=True)
                  for b in range(DEG_CH)]
            for cp in sc:
                cp.wait()
            return carry

        lax.fori_loop(0, DEG_NCH, body, 0)
        plsc.subcore_barrier()
        readback(acc, outref)

    @pl.when(c == 0)
    def _():
        chunk_pass(x0, acc, a0)
        chunk_pass(x1, acc, a1)
        deg_pass(acc, dga, 0)

    @pl.when(c == 1)
    def _():
        chunk_pass(x2, acc, a2)
        chunk_pass(x3, acc, a3)
        deg_pass(acc, dgb, ER // 2)


def _sc_call(x0, x1, x2, x3, srcr, dstr, zrows, ones):
    f32 = jnp.float32
    out_type = tuple(jax.ShapeDtypeStruct((NPAD, 32), f32) for _ in range(6))
    mesh = plsc.VectorSubcoreMesh(core_axis_name="c", subcore_axis_name="s")
    kern = pl.kernel(
        _sc_scatter, out_type=out_type, mesh=mesh,
        compiler_params=pltpu.CompilerParams(use_tc_tiling_on_sc=False),
        scratch_types=[
            pltpu.VMEM((CH * 128,), jnp.int32),
            pltpu.VMEM((CH, 128), jnp.int32),
            pltpu.VMEM((128, 32), f32),
            pltpu.VMEM((128, 32), f32),
            pltpu.VMEM((128, 32), f32),
            pltpu.VMEM((128, 32), f32),
            pltpu.VMEM((128, 32), f32),
            pltpu.VMEM((128, 32), f32),
            pltpu.VMEM_SHARED((NPAD, 32), f32),
            pltpu.SemaphoreType.DMA,
            pltpu.SemaphoreType.DMA,
        ],
    )
    return kern(x0, x1, x2, x3, srcr, dstr, zrows, ones)


_BLK_I = 2000
_BLK_N = 2000


def _lrelu(x):
    return jnp.where(x >= 0, x, 0.01 * x)


def _proj_body(f0, f1, w0, b0, w1, b1, p0, p1):
    a0 = jnp.dot(f0[...], w0[...], preferred_element_type=jnp.float32) + b0[...]
    a1 = jnp.dot(f1[...], w1[...], preferred_element_type=jnp.float32) + b1[...]
    p0[...] = _lrelu(a0)
    p1[...] = _lrelu(a1)


def _project(feat0, feat1, W0, b0, W1, b1):
    D0 = feat0.shape[1]
    D1 = feat1.shape[1]
    grid = NUM_ITEMS // _BLK_I
    return pl.pallas_call(
        _proj_body,
        grid=(grid,),
        in_specs=[
            pl.BlockSpec((_BLK_I, D0), lambda i: (i, 0)),
            pl.BlockSpec((_BLK_I, D1), lambda i: (i, 0)),
            pl.BlockSpec((D0, EMB), lambda i: (0, 0)),
            pl.BlockSpec((1, EMB), lambda i: (0, 0)),
            pl.BlockSpec((D1, EMB), lambda i: (0, 0)),
            pl.BlockSpec((1, EMB), lambda i: (0, 0)),
        ],
        out_specs=[
            pl.BlockSpec((_BLK_I, EMB), lambda i: (i, 0)),
            pl.BlockSpec((_BLK_I, EMB), lambda i: (i, 0)),
        ],
        out_shape=[
            jax.ShapeDtypeStruct((NUM_ITEMS, EMB), jnp.float32),
            jax.ShapeDtypeStruct((NUM_ITEMS, EMB), jnp.float32),
        ],
    )(feat0, feat1, W0, b0.reshape(1, EMB), W1, b1.reshape(1, EMB))


def _comb_body(a0, a1, a2, a3, dga, dgb, ne, wg0, wg1, wid, out):
    deg = jnp.maximum(dga[:, 0:1] + dgb[:, 0:1], 1.0)
    agg0 = jnp.concatenate([a0[...], a1[...]], axis=1) / deg
    agg1 = jnp.concatenate([a2[...], a3[...]], axis=1) / deg
    nev = ne[...]
    idp = jnp.dot(nev, wid[...], preferred_element_type=jnp.float32)
    h0 = _lrelu(jnp.dot(agg0, wg0[...], preferred_element_type=jnp.float32) + idp)
    h1 = _lrelu(jnp.dot(agg1, wg1[...], preferred_element_type=jnp.float32) + idp)
    out[...] = h0 + h1 + nev


def _combine(a0, a1, a2, a3, dga, dgb, node_emb, Wg0, Wg1, Wid):
    grid = N // _BLK_N
    cspec = pl.BlockSpec((_BLK_N, 32), lambda i: (i, 0))
    wspec = pl.BlockSpec((EMB, EMB), lambda i: (0, 0))
    return pl.pallas_call(
        _comb_body,
        grid=(grid,),
        in_specs=[cspec, cspec, cspec, cspec, cspec, cspec,
                  pl.BlockSpec((_BLK_N, EMB), lambda i: (i, 0)),
                  wspec, wspec, wspec],
        out_specs=pl.BlockSpec((_BLK_N, EMB), lambda i: (i, 0)),
        out_shape=jax.ShapeDtypeStruct((N, EMB), jnp.float32),
    )(a0, a1, a2, a3, dga, dgb, node_emb, Wg0, Wg1, Wid)


def kernel(node_emb, feat0, feat1, user_pref0, user_pref1, edge_index,
           W_proj0, b_proj0, W_proj1, b_proj1, W_gcn0, W_gcn1, W_id):
    p0, p1 = _project(feat0, feat1, W_proj0, b_proj0, W_proj1, b_proj1)
    x0 = jnp.concatenate([user_pref0, p0], axis=0)
    x1 = jnp.concatenate([user_pref1, p1], axis=0)
    pres = jnp.stack([x0, x1])

    npad_e = EPAD - E
    src = jnp.concatenate([edge_index[0], jnp.zeros((npad_e,), jnp.int32)])
    dst = jnp.concatenate([edge_index[1],
                           jnp.full((npad_e,), N, jnp.int32)])
    dstr = dst.reshape(ER, 128)

    zrows = jnp.zeros((NPAD, 32), jnp.float32)
    ones = jnp.ones((128, 32), jnp.float32)

    a0, a1, a2, a3, dga, dgb = _sc_call(
        x0[:, :32], x0[:, 32:], x1[:, :32], x1[:, 32:],
        src, dstr, zrows, ones)

    emb = _combine(a0, a1, a2, a3, dga, dgb, node_emb, W_gcn0, W_gcn1, W_id)
    return emb[:NUM_USERS], emb[NUM_USERS:], node_emb, pres


# speedup vs baseline: 4.7144x; 1.0150x over previous
import jax
import jax.numpy as jnp
from jax import lax
from jax.experimental import pallas as pl
from jax.experimental.pallas import tpu as pltpu
from jax.experimental.pallas import tpu_sc as plsc

NUM_USERS = 10000
NUM_ITEMS = 40000
N = NUM_USERS + NUM_ITEMS
EMB = 64
E = 800000

W = 16
NPAD = 50176
RPT = NPAD // 16
EPAD = 819200
ER = EPAD // 128
ROWS_TILE = ER // 16
CH = 16
NCH = ROWS_TILE // CH
GB = CH // 2
DEG_CH = 8
DEG_ROWS_TILE = (ER // 2) // 16
DEG_NCH = DEG_ROWS_TILE // DEG_CH
NBUF = 3


def _sc_scatter(x0, x1, x2, x3, x4, x5, x6, x7, srcf, dstr, zrows, ones,
                a0, a1, a2, a3, a4, a5, a6, a7, dga, dgb,
                src_v, dst_v, r0_v, r1_v, r2_v, table, acc,
                gsem, ssem):
    c = lax.axis_index("c")
    s = lax.axis_index("s")
    rows = [r0_v, r1_v, r2_v]

    def zero_acc():
        pltpu.sync_copy(zrows.at[pl.ds(s * RPT, RPT)],
                        acc.at[pl.ds(s * RPT, RPT)])

    def readback(outref):
        pltpu.sync_copy(acc.at[pl.ds(s * RPT, RPT)],
                        outref.at[pl.ds(s * RPT, RPT)])

    def chunk_pass(xref, outref):
        pltpu.sync_copy(xref.at[pl.ds(s * RPT, RPT)],
                        table.at[pl.ds(s * RPT, RPT)])
        zero_acc()
        plsc.subcore_barrier()
        base = s * ROWS_TILE

        def body(i, carry):
            r0 = base + i * CH
            pltpu.sync_copy(srcf.at[pl.ds(r0 * 128, CH * 128)], src_v)
            pltpu.sync_copy(dstr.at[pl.ds(r0, CH)], dst_v)
            g = {}
            sc = {}

            def fire_scatters(b):
                g[b].wait()
                buf = rows[b % NBUF]
                for h in range(2):
                    sc[(b, h)] = pltpu.async_copy(
                        buf.at[pl.ds(h * 128, 128)],
                        acc.at[dst_v.at[2 * b + h]], ssem, add=True)

            for b in range(GB):
                if b >= NBUF:
                    sc[(b - NBUF, 0)].wait()
                    sc[(b - NBUF, 1)].wait()
                g[b] = pltpu.async_copy(
                    table.at[src_v.at[pl.ds(b * 256, 256)]],
                    rows[b % NBUF], gsem)
                if b >= 1:
                    fire_scatters(b - 1)
            fire_scatters(GB - 1)
            for b in range(GB - NBUF, GB):
                sc[(b, 0)].wait()
                sc[(b, 1)].wait()
            return carry

        lax.fori_loop(0, NCH, body, 0)
        plsc.subcore_barrier()
        readback(outref)

    def deg_pass(outref, lo):
        zero_acc()
        pltpu.sync_copy(ones, r0_v.at[pl.ds(0, 128)])
        plsc.subcore_barrier()
        base = lo + s * DEG_ROWS_TILE

        def body(i, carry):
            r0 = base + i * DEG_CH
            pltpu.sync_copy(dstr.at[pl.ds(r0, DEG_CH)],
                            dst_v.at[pl.ds(0, DEG_CH)])
            sc = [pltpu.async_copy(r0_v.at[pl.ds(0, 128)],
                                   acc.at[dst_v.at[b]], ssem, add=True)
                  for b in range(DEG_CH)]
            for cp in sc:
                cp.wait()
            return carry

        lax.fori_loop(0, DEG_NCH, body, 0)
        plsc.subcore_barrier()
        readback(outref)

    @pl.when(c == 0)
    def _():
        chunk_pass(x0, a0)
        chunk_pass(x1, a1)
        chunk_pass(x2, a2)
        chunk_pass(x3, a3)
        deg_pass(dga, 0)

    @pl.when(c == 1)
    def _():
        chunk_pass(x4, a4)
        chunk_pass(x5, a5)
        chunk_pass(x6, a6)
        chunk_pass(x7, a7)
        deg_pass(dgb, ER // 2)


def _sc_call(xc, srcf, dstr, zrows, ones):
    f32 = jnp.float32
    out_type = tuple(jax.ShapeDtypeStruct((NPAD, W), f32) for _ in range(10))
    mesh = plsc.VectorSubcoreMesh(core_axis_name="c", subcore_axis_name="s")
    kern = pl.kernel(
        _sc_scatter, out_type=out_type, mesh=mesh,
        compiler_params=pltpu.CompilerParams(use_tc_tiling_on_sc=False),
        scratch_types=[
            pltpu.VMEM((CH * 128,), jnp.int32),
            pltpu.VMEM((CH, 128), jnp.int32),
            pltpu.VMEM((256, W), f32),
            pltpu.VMEM((256, W), f32),
            pltpu.VMEM((256, W), f32),
            pltpu.VMEM_SHARED((NPAD, W), f32),
            pltpu.VMEM_SHARED((NPAD, W), f32),
            pltpu.SemaphoreType.DMA,
            pltpu.SemaphoreType.DMA,
        ],
    )
    return kern(*xc, srcf, dstr, zrows, ones)


_BLK_I = 2000
_BLK_N = 2000


def _lrelu(x):
    return jnp.where(x >= 0, x, 0.01 * x)


def _proj_body(f0, f1, w0, b0, w1, b1, p0, p1):
    a0 = jnp.dot(f0[...], w0[...], preferred_element_type=jnp.float32) + b0[...]
    a1 = jnp.dot(f1[...], w1[...], preferred_element_type=jnp.float32) + b1[...]
    p0[...] = _lrelu(a0)
    p1[...] = _lrelu(a1)


def _project(feat0, feat1, W0, b0, W1, b1):
    D0 = feat0.shape[1]
    D1 = feat1.shape[1]
    grid = NUM_ITEMS // _BLK_I
    return pl.pallas_call(
        _proj_body,
        grid=(grid,),
        in_specs=[
            pl.BlockSpec((_BLK_I, D0), lambda i: (i, 0)),
            pl.BlockSpec((_BLK_I, D1), lambda i: (i, 0)),
            pl.BlockSpec((D0, EMB), lambda i: (0, 0)),
            pl.BlockSpec((1, EMB), lambda i: (0, 0)),
            pl.BlockSpec((D1, EMB), lambda i: (0, 0)),
            pl.BlockSpec((1, EMB), lambda i: (0, 0)),
        ],
        out_specs=[
            pl.BlockSpec((_BLK_I, EMB), lambda i: (i, 0)),
            pl.BlockSpec((_BLK_I, EMB), lambda i: (i, 0)),
        ],
        out_shape=[
            jax.ShapeDtypeStruct((NUM_ITEMS, EMB), jnp.float32),
            jax.ShapeDtypeStruct((NUM_ITEMS, EMB), jnp.float32),
        ],
    )(feat0, feat1, W0, b0.reshape(1, EMB), W1, b1.reshape(1, EMB))


def _comb_body(a0, a1, a2, a3, a4, a5, a6, a7, dga, dgb, ne, wg0, wg1, wid,
               out):
    deg = jnp.maximum(dga[:, 0:1] + dgb[:, 0:1], 1.0)
    agg0 = jnp.concatenate([a0[...], a1[...], a2[...], a3[...]], axis=1) / deg
    agg1 = jnp.concatenate([a4[...], a5[...], a6[...], a7[...]], axis=1) / deg
    nev = ne[...]
    idp = jnp.dot(nev, wid[...], preferred_element_type=jnp.float32)
    h0 = _lrelu(jnp.dot(agg0, wg0[...], preferred_element_type=jnp.float32) + idp)
    h1 = _lrelu(jnp.dot(agg1, wg1[...], preferred_element_type=jnp.float32) + idp)
    out[...] = h0 + h1 + nev


def _combine(aggs, dga, dgb, node_emb, Wg0, Wg1, Wid):
    grid = N // _BLK_N
    cspec = pl.BlockSpec((_BLK_N, W), lambda i: (i, 0))
    wspec = pl.BlockSpec((EMB, EMB), lambda i: (0, 0))
    return pl.pallas_call(
        _comb_body,
        grid=(grid,),
        in_specs=[cspec] * 10 + [
            pl.BlockSpec((_BLK_N, EMB), lambda i: (i, 0)),
            wspec, wspec, wspec],
        out_specs=pl.BlockSpec((_BLK_N, EMB), lambda i: (i, 0)),
        out_shape=jax.ShapeDtypeStruct((N, EMB), jnp.float32),
    )(*aggs, dga, dgb, node_emb, Wg0, Wg1, Wid)


def kernel(node_emb, feat0, feat1, user_pref0, user_pref1, edge_index,
           W_proj0, b_proj0, W_proj1, b_proj1, W_gcn0, W_gcn1, W_id):
    p0, p1 = _project(feat0, feat1, W_proj0, b_proj0, W_proj1, b_proj1)
    x0 = jnp.concatenate([user_pref0, p0], axis=0)
    x1 = jnp.concatenate([user_pref1, p1], axis=0)
    pres = jnp.stack([x0, x1])

    npad_e = EPAD - E
    src = jnp.concatenate([edge_index[0], jnp.zeros((npad_e,), jnp.int32)])
    dst = jnp.concatenate([edge_index[1],
                           jnp.full((npad_e,), N, jnp.int32)])
    dstr = dst.reshape(ER, 128)

    zrows = jnp.zeros((NPAD, W), jnp.float32)
    ones = jnp.ones((128, W), jnp.float32)
    rpad = jnp.zeros((NPAD - N, W), jnp.float32)
    xc = [jnp.concatenate([x[:, k * W:(k + 1) * W], rpad], axis=0)
          for x in (x0, x1) for k in range(4)]

    outs = _sc_call(xc, src, dstr, zrows, ones)
    aggs, dga, dgb = outs[:8], outs[8], outs[9]

    emb = _combine(aggs, dga, dgb, node_emb, W_gcn0, W_gcn1, W_id)
    return emb[:NUM_USERS], emb[NUM_USERS:], node_emb, pres
